# Initial kernel scaffold; baseline (speedup 1.0000x reference)
import jax
import jax.numpy as jnp
from jax.experimental import pallas as pl


def kernel(x, weight, offsets):
    raise NotImplementedError("write your pallas kernel here")


# speedup vs baseline: 2.2020x; 2.2020x over previous
import jax
import jax.numpy as jnp
from jax.experimental import pallas as pl
from jax.experimental.pallas import tpu as pltpu


def _body(x_ref, s_ref, ge_ref, o_ref):
    xb = x_ref[...].astype(jnp.bfloat16)
    expanded = jnp.dot(xb, s_ref[...], preferred_element_type=jnp.float32)
    o_ref[...] = expanded * ge_ref[...]


def kernel(x, weight, offsets):
    B, D = x.shape
    E = weight.shape[1]
    DE = D * E

    ge = weight[offsets].reshape(1, DE).astype(jnp.float32)
    sel = (jnp.arange(DE, dtype=jnp.int32)[None, :] // E
           == jnp.arange(D, dtype=jnp.int32)[:, None])
    s = sel.astype(jnp.bfloat16)

    tb = 2048
    if B % tb != 0:
        tb = max(8, min(tb, B))
    grid = (pl.cdiv(B, tb),)

    return pl.pallas_call(
        _body,
        out_shape=jax.ShapeDtypeStruct((B, DE), jnp.float32),
        grid=grid,
        in_specs=[
            pl.BlockSpec((tb, D), lambda i: (i, 0)),
            pl.BlockSpec((D, DE), lambda i: (0, 0)),
            pl.BlockSpec((1, DE), lambda i: (0, 0)),
        ],
        out_specs=pl.BlockSpec((tb, DE), lambda i: (i, 0)),
        compiler_params=pltpu.CompilerParams(
            dimension_semantics=("parallel",),
        ),
        cost_estimate=pl.CostEstimate(
            flops=2 * B * D * DE,
            transcendentals=0,
            bytes_accessed=4 * (B * DE + B * D + D * DE + DE),
        ),
    )(x, s, ge)


# speedup vs baseline: 2.3095x; 1.0488x over previous
import jax
import jax.numpy as jnp
from jax.experimental import pallas as pl
from jax.experimental.pallas import tpu as pltpu


def _body(x_ref, s_ref, ge_ref, o_ref):
    xb = x_ref[...].astype(jnp.bfloat16)
    expanded = jnp.dot(xb, s_ref[...], preferred_element_type=jnp.float32)
    o_ref[...] = expanded * ge_ref[...]


def kernel(x, weight, offsets):
    B, D = x.shape
    E = weight.shape[1]
    DE = D * E

    ge = weight[offsets].reshape(1, DE).astype(jnp.float32)
    sel = (jnp.arange(DE, dtype=jnp.int32)[None, :] // E
           == jnp.arange(D, dtype=jnp.int32)[:, None])
    s = sel.astype(jnp.bfloat16)

    tb = 4096
    if B % tb != 0:
        tb = max(8, min(tb, B))
    grid = (pl.cdiv(B, tb),)

    return pl.pallas_call(
        _body,
        out_shape=jax.ShapeDtypeStruct((B, DE), jnp.float32),
        grid=grid,
        in_specs=[
            pl.BlockSpec((tb, D), lambda i: (i, 0)),
            pl.BlockSpec((D, DE), lambda i: (0, 0)),
            pl.BlockSpec((1, DE), lambda i: (0, 0)),
        ],
        out_specs=pl.BlockSpec((tb, DE), lambda i: (i, 0)),
        compiler_params=pltpu.CompilerParams(
            dimension_semantics=("parallel",),
        ),
        cost_estimate=pl.CostEstimate(
            flops=2 * B * D * DE,
            transcendentals=0,
            bytes_accessed=4 * (B * DE + B * D + D * DE + DE),
        ),
    )(x, s, ge)


# speedup vs baseline: 2.3372x; 1.0120x over previous
import jax
import jax.numpy as jnp
from jax.experimental import pallas as pl
from jax.experimental.pallas import tpu as pltpu


def _body(x_ref, s_ref, ge_ref, o_ref):
    xb = x_ref[...].astype(jnp.bfloat16)
    expanded = jnp.dot(xb, s_ref[...], preferred_element_type=jnp.float32)
    o_ref[...] = expanded * ge_ref[...]


def kernel(x, weight, offsets):
    B, D = x.shape
    E = weight.shape[1]
    DE = D * E

    ge = weight[offsets].reshape(1, DE).astype(jnp.float32)
    sel = (jnp.arange(DE, dtype=jnp.int32)[None, :] // E
           == jnp.arange(D, dtype=jnp.int32)[:, None])
    s = sel.astype(jnp.bfloat16)

    tb = 8192
    if B % tb != 0:
        tb = max(8, min(tb, B))
    grid = (pl.cdiv(B, tb),)

    return pl.pallas_call(
        _body,
        out_shape=jax.ShapeDtypeStruct((B, DE), jnp.float32),
        grid=grid,
        in_specs=[
            pl.BlockSpec((tb, D), lambda i: (i, 0)),
            pl.BlockSpec((D, DE), lambda i: (0, 0)),
            pl.BlockSpec((1, DE), lambda i: (0, 0)),
        ],
        out_specs=pl.BlockSpec((tb, DE), lambda i: (i, 0)),
        compiler_params=pltpu.CompilerParams(
            dimension_semantics=("parallel",),
        ),
        cost_estimate=pl.CostEstimate(
            flops=2 * B * D * DE,
            transcendentals=0,
            bytes_accessed=4 * (B * DE + B * D + D * DE + DE),
        ),
    )(x, s, ge)


# speedup vs baseline: 3.4606x; 1.4807x over previous
import jax
import jax.numpy as jnp
from jax import lax
from jax.experimental import pallas as pl
from jax.experimental.pallas import tpu as pltpu


def _body(xt_ref, s_ref, ge_ref, o_ref):
    xb = xt_ref[...].astype(jnp.bfloat16)
    expanded = lax.dot_general(
        xb, s_ref[...],
        dimension_numbers=(((0,), (0,)), ((), ())),
        preferred_element_type=jnp.float32,
    )
    o_ref[...] = expanded * ge_ref[...]


def kernel(x, weight, offsets):
    B, D = x.shape
    E = weight.shape[1]
    DE = D * E

    ge = weight[offsets].reshape(1, DE).astype(jnp.float32)
    sel = (jnp.arange(DE, dtype=jnp.int32)[None, :] // E
           == jnp.arange(D, dtype=jnp.int32)[:, None])
    s = sel.astype(jnp.bfloat16)

    tb = 8192
    if B % tb != 0:
        tb = max(8, min(tb, B))
    grid = (pl.cdiv(B, tb),)

    return pl.pallas_call(
        _body,
        out_shape=jax.ShapeDtypeStruct((B, DE), jnp.float32),
        grid=grid,
        in_specs=[
            pl.BlockSpec((D, tb), lambda i: (0, i)),
            pl.BlockSpec((D, DE), lambda i: (0, 0)),
            pl.BlockSpec((1, DE), lambda i: (0, 0)),
        ],
        out_specs=pl.BlockSpec((tb, DE), lambda i: (i, 0)),
        compiler_params=pltpu.CompilerParams(
            dimension_semantics=("parallel",),
        ),
        cost_estimate=pl.CostEstimate(
            flops=2 * B * D * DE,
            transcendentals=0,
            bytes_accessed=4 * (B * DE + B * D + D * DE + DE),
        ),
    )(x.T, s, ge)


# speedup vs baseline: 3.4823x; 1.0062x over previous
import numpy as np

import jax
import jax.numpy as jnp
from jax import lax
from jax.experimental import pallas as pl
from jax.experimental.pallas import tpu as pltpu


def _body(xt_ref, s_ref, ge_ref, o_ref):
    xb = xt_ref[...].astype(jnp.bfloat16)
    expanded = lax.dot_general(
        xb, s_ref[...],
        dimension_numbers=(((0,), (0,)), ((), ())),
        preferred_element_type=jnp.float32,
    )
    o_ref[...] = expanded * ge_ref[...]


def kernel(x, weight, offsets):
    B, D = x.shape
    E = weight.shape[1]
    DE = D * E

    ge = weight.T[:, offsets].T.reshape(1, DE).astype(jnp.float32)
    sel = (np.arange(DE, dtype=np.int32)[None, :] // E
           == np.arange(D, dtype=np.int32)[:, None])
    s = jnp.asarray(sel.astype(np.float32), dtype=jnp.bfloat16)

    tb = 8192
    if B % tb != 0:
        tb = max(8, min(tb, B))
    grid = (pl.cdiv(B, tb),)

    return pl.pallas_call(
        _body,
        out_shape=jax.ShapeDtypeStruct((B, DE), jnp.float32),
        grid=grid,
        in_specs=[
            pl.BlockSpec((D, tb), lambda i: (0, i)),
            pl.BlockSpec((D, DE), lambda i: (0, 0)),
            pl.BlockSpec((1, DE), lambda i: (0, 0)),
        ],
        out_specs=pl.BlockSpec((tb, DE), lambda i: (i, 0)),
        compiler_params=pltpu.CompilerParams(
            dimension_semantics=("arbitrary",),
        ),
        cost_estimate=pl.CostEstimate(
            flops=2 * B * D * DE,
            transcendentals=0,
            bytes_accessed=4 * (B * DE + B * D + D * DE + DE),
        ),
    )(x.T, s, ge)
